# baseline (speedup 1.0000x reference)
import jax
import jax.numpy as jnp
from jax import lax
from jax.experimental import pallas as pl
from jax.experimental.pallas import tpu as pltpu
from jax.experimental.pallas import tpu_sc as plsc

B = 16384
D = 128
H = 128
M = 100000

NC = 2
NS = 16
NW = NC * NS
CHUNK = B // NW
SUB = 128
NSUB = CHUNK // SUB
STRIP = 3200
MPAD = NW * STRIP
NVREG = B // 16
PBITS = 14

_f32 = jnp.float32
_i32 = jnp.int32


def _sc_mesh():
    return plsc.VectorSubcoreMesh(core_axis_name="c", subcore_axis_name="s")


def _wid():
    return lax.axis_index("s") * NC + lax.axis_index("c")


def _sc1_body(memk_hbm, memv_hbm, idx_hbm,
              stored_k_hbm, stored_v_hbm, postbl_hbm,
              idxall_v, kb0, vb0, kb1, vb1, strip_v,
              semg, semw):
    wid = _wid()
    pltpu.sync_copy(idx_hbm, idxall_v)
    kb = (kb0, kb1)
    vb = (vb0, vb1)

    def fire_gather(p):
        base = (wid * NSUB + p) * SUB
        il = idxall_v.at[pl.ds(base, SUB)]
        return (pltpu.async_copy(memk_hbm.at[il], kb[p % 2], semg),
                pltpu.async_copy(memv_hbm.at[il], vb[p % 2], semg))

    def fire_wb(p):
        base = (wid * NSUB + p) * SUB
        return (pltpu.async_copy(kb[p % 2], stored_k_hbm.at[pl.ds(base, SUB)], semw),
                pltpu.async_copy(vb[p % 2], stored_v_hbm.at[pl.ds(base, SUB)], semw))

    lanes = lax.broadcasted_iota(_i32, (16,), 0)
    nxt = jnp.minimum(lanes + 1, 15)
    strip_lo = wid * STRIP

    UNROLL = 4

    def win_it(tb, _):
        for u in range(UNROLL):
            t = tb * UNROLL + u
            iv = idxall_v[pl.ds(t * 16, 16)]
            key = (iv << PBITS) | (t * 16 + lanes)
            key = lax.sort(key)
            nkey = jnp.take_along_axis(key, nxt, axis=0)
            slot = key >> PBITS
            keep = (lanes == 15) | (slot != (nkey >> PBITS))
            loc = slot - strip_lo
            inr = (loc >= 0) & (loc < STRIP)
            safe = jnp.where(inr, loc, 0)
            plsc.store_scatter(strip_v, [safe], key & (2**PBITS - 1),
                               mask=keep & inr)
        return _

    WSLICE = NVREG // UNROLL // NSUB
    g = fire_gather(0)
    wbs = []
    for p in range(NSUB):
        lax.fori_loop(p * WSLICE, (p + 1) * WSLICE, win_it, 0)
        g[0].wait()
        g[1].wait()
        if p + 1 < NSUB:
            if p >= 1:
                wbs[p - 1][0].wait()
                wbs[p - 1][1].wait()
            g = fire_gather(p + 1)
        wbs.append(fire_wb(p))

    pltpu.sync_copy(strip_v, postbl_hbm.at[pl.ds(strip_lo, STRIP)])
    wbs[-2][0].wait()
    wbs[-2][1].wait()
    wbs[-1][0].wait()
    wbs[-1][1].wait()


def _sc1(mem_keys, mem_vals, idx):
    kern = pl.kernel(
        _sc1_body,
        out_type=(
            jax.ShapeDtypeStruct((B, D), _f32),
            jax.ShapeDtypeStruct((B, D), _f32),
            jax.ShapeDtypeStruct((MPAD,), _i32),
        ),
        mesh=_sc_mesh(),
        compiler_params=pltpu.CompilerParams(needs_layout_passes=False),
        cost_estimate=pl.CostEstimate(
            flops=0, transcendentals=0, bytes_accessed=40_000_000),
        scratch_types=(
            pltpu.VMEM((B,), _i32),
            pltpu.VMEM((SUB, D), _f32),
            pltpu.VMEM((SUB, D), _f32),
            pltpu.VMEM((SUB, D), _f32),
            pltpu.VMEM((SUB, D), _f32),
            pltpu.VMEM((STRIP,), _i32),
            pltpu.SemaphoreType.DMA,
            pltpu.SemaphoreType.DMA,
        ),
    )
    return kern(mem_keys, mem_vals, idx)


def _sc2_body(idx2_hbm, postbl_hbm, k_hbm, v_hbm, nk_hbm, nv_hbm,
              idxc_v, wq_v, kb0, vb0, kb1, vb1, semw, semg, sems):
    wid = _wid()
    kb = (kb0, kb1)
    vb = (vb0, vb1)

    pltpu.sync_copy(idx2_hbm.at[pl.ds(wid * NSUB, NSUB)], idxc_v)
    wq = [pltpu.async_copy(postbl_hbm.at[idxc_v.at[p]], wq_v.at[p], semw)
          for p in range(NSUB)]
    for c in wq:
        c.wait()

    def fire_gather(p):
        return (pltpu.async_copy(k_hbm.at[wq_v.at[p]], kb[p % 2], semg),
                pltpu.async_copy(v_hbm.at[wq_v.at[p]], vb[p % 2], semg))

    def fire_scatter(p):
        return (pltpu.async_copy(kb[p % 2], nk_hbm.at[idxc_v.at[p]], sems),
                pltpu.async_copy(vb[p % 2], nv_hbm.at[idxc_v.at[p]], sems))

    g = fire_gather(0)
    scs = []
    for p in range(NSUB):
        g[0].wait()
        g[1].wait()
        if p + 1 < NSUB:
            if p >= 1:
                scs[p - 1][0].wait()
                scs[p - 1][1].wait()
            g = fire_gather(p + 1)
        scs.append(fire_scatter(p))
    scs[-2][0].wait()
    scs[-2][1].wait()
    scs[-1][0].wait()
    scs[-1][1].wait()


def _sc2(idx2, postbl, k, v, nk_ref, nv_ref):
    kern = pl.kernel(
        _sc2_body,
        out_type=(),
        mesh=_sc_mesh(),
        compiler_params=pltpu.CompilerParams(needs_layout_passes=False),
        cost_estimate=pl.CostEstimate(
            flops=0, transcendentals=0, bytes_accessed=34_000_000),
        scratch_types=(
            pltpu.VMEM((NSUB, SUB), _i32),
            pltpu.VMEM((NSUB, SUB), _i32),
            pltpu.VMEM((SUB, D), _f32),
            pltpu.VMEM((SUB, D), _f32),
            pltpu.VMEM((SUB, D), _f32),
            pltpu.VMEM((SUB, D), _f32),
            pltpu.SemaphoreType.DMA,
            pltpu.SemaphoreType.DMA,
            pltpu.SemaphoreType.DMA,
        ),
    )
    kern(idx2, postbl, k, v, nk_ref, nv_ref)


RB = 1024


def _tca_body(s_ref, wk_ref, bk_ref, wv_ref, bv_ref, k_ref, v_ref):
    s = s_ref[...]
    k_ref[...] = jnp.dot(s, wk_ref[...], preferred_element_type=_f32) + bk_ref[...]
    v_ref[...] = jnp.dot(s, wv_ref[...], preferred_element_type=_f32) + bv_ref[...]


def _tca(S_t, Wk, bk, Wv, bv):
    row = lambda i: (i, 0)
    full = lambda i: (0, 0)
    rspec = pl.BlockSpec((RB, D), row)
    return pl.pallas_call(
        _tca_body,
        grid=(B // RB,),
        in_specs=[
            rspec,
            pl.BlockSpec((D, D), full), pl.BlockSpec((1, D), full),
            pl.BlockSpec((D, D), full), pl.BlockSpec((1, D), full),
        ],
        out_specs=[rspec, rspec],
        out_shape=[
            jax.ShapeDtypeStruct((B, D), _f32),
            jax.ShapeDtypeStruct((B, D), _f32),
        ],
    )(S_t, Wk, bk.reshape(1, D), Wv, bv.reshape(1, D))


def _tcc_body(s_ref, k_ref, sk_ref, sv_ref, w1a_ref, w1b_ref, b1_ref,
              w2_ref, b2_ref, d_ref):
    s = s_ref[...]
    diff = k_ref[...] - sk_ref[...]
    match = jnp.exp(-jnp.mean(diff * diff, axis=-1, keepdims=True))
    rv = sv_ref[...] * match
    h = jnp.dot(s, w1a_ref[...], preferred_element_type=_f32)
    h = h + jnp.dot(rv, w1b_ref[...], preferred_element_type=_f32)
    h = jax.nn.relu(h + b1_ref[...])
    d_ref[...] = jnp.dot(h, w2_ref[...], preferred_element_type=_f32) + b2_ref[...]


def _tcc(S_t, k, stored_k, stored_v, W1, b1, W2, b2):
    w1a, w1b = W1[:D], W1[D:]
    row = lambda i: (i, 0)
    full = lambda i: (0, 0)
    rspec = pl.BlockSpec((RB, D), row)
    return pl.pallas_call(
        _tcc_body,
        grid=(B // RB,),
        in_specs=[
            rspec, rspec, rspec, rspec,
            pl.BlockSpec((D, H), full), pl.BlockSpec((D, H), full),
            pl.BlockSpec((1, H), full),
            pl.BlockSpec((H, D), full), pl.BlockSpec((1, D), full),
        ],
        out_specs=rspec,
        out_shape=jax.ShapeDtypeStruct((B, D), _f32),
    )(S_t, k, stored_k, stored_v, w1a, w1b, b1.reshape(1, H),
      W2, b2.reshape(1, D))


RT = 5000


def _tcb_body(mk_ref, mv_ref, nk_ref, nv_ref):
    nk_ref[...] = mk_ref[...]
    nv_ref[...] = mv_ref[...]


def _tcb(mem_keys, mem_vals):
    row = lambda i: (i, 0)
    rspec = pl.BlockSpec((RT, D), row)
    return pl.pallas_call(
        _tcb_body,
        grid=(M // RT,),
        in_specs=[rspec, rspec],
        out_specs=[rspec, rspec],
        out_shape=[
            jax.ShapeDtypeStruct((M, D), _f32),
            jax.ShapeDtypeStruct((M, D), _f32),
        ],
    )(mem_keys, mem_vals)


@jax.jit
def kernel(S_t, mem_keys, mem_vals, Wk, bk, Wv, bv, W1, b1, W2, b2, idx):
    stored_k, stored_v, postbl = _sc1(mem_keys, mem_vals, idx)
    nk, nv = _tcb(mem_keys, mem_vals)
    k, v = _tca(S_t, Wk, bk, Wv, bv)
    nk_ref = jax.new_ref(nk)
    nv_ref = jax.new_ref(nv)
    idx2 = idx.reshape(B // SUB, SUB)
    _sc2(idx2, postbl, k, v, nk_ref, nv_ref)
    delta = _tcc(S_t, k, stored_k, stored_v, W1, b1, W2, b2)
    return delta, nk_ref[...], nv_ref[...]


# speedup vs baseline: 1.0055x; 1.0055x over previous
import jax
import jax.numpy as jnp
from jax import lax
from jax.experimental import pallas as pl
from jax.experimental.pallas import tpu as pltpu
from jax.experimental.pallas import tpu_sc as plsc

B = 16384
D = 128
H = 128
M = 100000

NC = 2
NS = 16
NW = NC * NS
CHUNK = B // NW
SUB = 128
NSUB = CHUNK // SUB
STRIP = 3200
MPAD = NW * STRIP
NVREG = B // 16
PBITS = 14

_f32 = jnp.float32
_i32 = jnp.int32


def _sc_mesh():
    return plsc.VectorSubcoreMesh(core_axis_name="c", subcore_axis_name="s")


def _wid():
    return lax.axis_index("s") * NC + lax.axis_index("c")


def _sc1_body(memk_hbm, memv_hbm, idx_hbm,
              stored_k_hbm, stored_v_hbm, postbl_hbm,
              idxall_v, kb0, vb0, kb1, vb1, strip_v,
              semg, semw):
    wid = _wid()
    pltpu.sync_copy(idx_hbm, idxall_v)
    kb = (kb0, kb1)
    vb = (vb0, vb1)

    def fire_gather(p):
        base = (wid * NSUB + p) * SUB
        il = idxall_v.at[pl.ds(base, SUB)]
        return (pltpu.async_copy(memk_hbm.at[il], kb[p % 2], semg),
                pltpu.async_copy(memv_hbm.at[il], vb[p % 2], semg))

    def fire_wb(p):
        base = (wid * NSUB + p) * SUB
        return (pltpu.async_copy(kb[p % 2], stored_k_hbm.at[pl.ds(base, SUB)], semw),
                pltpu.async_copy(vb[p % 2], stored_v_hbm.at[pl.ds(base, SUB)], semw))

    lanes = lax.broadcasted_iota(_i32, (16,), 0)
    nxt = jnp.minimum(lanes + 1, 15)
    strip_lo = wid * STRIP

    UNROLL = 4

    def win_it(tb, _):
        for u in range(UNROLL):
            t = tb * UNROLL + u
            iv = idxall_v[pl.ds(t * 16, 16)]
            key = (iv << PBITS) | (t * 16 + lanes)
            key = lax.sort(key)
            nkey = jnp.take_along_axis(key, nxt, axis=0)
            slot = key >> PBITS
            keep = (lanes == 15) | (slot != (nkey >> PBITS))
            loc = slot - strip_lo
            inr = (loc >= 0) & (loc < STRIP)
            safe = jnp.where(inr, loc, 0)
            plsc.store_scatter(strip_v, [safe], key & (2**PBITS - 1),
                               mask=keep & inr)
        return _

    WSLICE = NVREG // UNROLL // NSUB
    g = fire_gather(0)
    wbs = []
    for p in range(NSUB):
        lax.fori_loop(p * WSLICE, (p + 1) * WSLICE, win_it, 0)
        g[0].wait()
        g[1].wait()
        if p + 1 < NSUB:
            if p >= 1:
                wbs[p - 1][0].wait()
                wbs[p - 1][1].wait()
            g = fire_gather(p + 1)
        wbs.append(fire_wb(p))

    pltpu.sync_copy(strip_v, postbl_hbm.at[pl.ds(strip_lo, STRIP)])
    wbs[-2][0].wait()
    wbs[-2][1].wait()
    wbs[-1][0].wait()
    wbs[-1][1].wait()


def _sc1(mem_keys, mem_vals, idx):
    kern = pl.kernel(
        _sc1_body,
        out_type=(
            jax.ShapeDtypeStruct((B, D), _f32),
            jax.ShapeDtypeStruct((B, D), _f32),
            jax.ShapeDtypeStruct((MPAD,), _i32),
        ),
        mesh=_sc_mesh(),
        compiler_params=pltpu.CompilerParams(needs_layout_passes=False),
        cost_estimate=pl.CostEstimate(
            flops=0, transcendentals=0, bytes_accessed=40_000_000),
        scratch_types=(
            pltpu.VMEM((B,), _i32),
            pltpu.VMEM((SUB, D), _f32),
            pltpu.VMEM((SUB, D), _f32),
            pltpu.VMEM((SUB, D), _f32),
            pltpu.VMEM((SUB, D), _f32),
            pltpu.VMEM((STRIP,), _i32),
            pltpu.SemaphoreType.DMA,
            pltpu.SemaphoreType.DMA,
        ),
    )
    return kern(mem_keys, mem_vals, idx)


def _sc2_body(idx2_hbm, postbl_hbm, k_hbm, v_hbm, nk_hbm, nv_hbm,
              idxc_v, wq_v, kb0, vb0, kb1, vb1, semw, semg, sems):
    wid = _wid()
    kb = (kb0, kb1)
    vb = (vb0, vb1)

    pltpu.sync_copy(idx2_hbm.at[pl.ds(wid * NSUB, NSUB)], idxc_v)
    wq = [pltpu.async_copy(postbl_hbm.at[idxc_v.at[p]], wq_v.at[p], semw)
          for p in range(NSUB)]
    for c in wq:
        c.wait()

    def fire_gather(p):
        return (pltpu.async_copy(k_hbm.at[wq_v.at[p]], kb[p % 2], semg),
                pltpu.async_copy(v_hbm.at[wq_v.at[p]], vb[p % 2], semg))

    def fire_scatter(p):
        return (pltpu.async_copy(kb[p % 2], nk_hbm.at[idxc_v.at[p]], sems),
                pltpu.async_copy(vb[p % 2], nv_hbm.at[idxc_v.at[p]], sems))

    g = fire_gather(0)
    scs = []
    for p in range(NSUB):
        g[0].wait()
        g[1].wait()
        if p + 1 < NSUB:
            if p >= 1:
                scs[p - 1][0].wait()
                scs[p - 1][1].wait()
            g = fire_gather(p + 1)
        scs.append(fire_scatter(p))
    scs[-2][0].wait()
    scs[-2][1].wait()
    scs[-1][0].wait()
    scs[-1][1].wait()


def _sc2(idx2, postbl, k, v, nk_ref, nv_ref):
    kern = pl.kernel(
        _sc2_body,
        out_type=(),
        mesh=_sc_mesh(),
        compiler_params=pltpu.CompilerParams(needs_layout_passes=False),
        cost_estimate=pl.CostEstimate(
            flops=0, transcendentals=0, bytes_accessed=34_000_000),
        scratch_types=(
            pltpu.VMEM((NSUB, SUB), _i32),
            pltpu.VMEM((NSUB, SUB), _i32),
            pltpu.VMEM((SUB, D), _f32),
            pltpu.VMEM((SUB, D), _f32),
            pltpu.VMEM((SUB, D), _f32),
            pltpu.VMEM((SUB, D), _f32),
            pltpu.SemaphoreType.DMA,
            pltpu.SemaphoreType.DMA,
            pltpu.SemaphoreType.DMA,
        ),
    )
    kern(idx2, postbl, k, v, nk_ref, nv_ref)


RB = 1024


def _tca_body(s_ref, wk_ref, bk_ref, wv_ref, bv_ref, k_ref, v_ref):
    s = s_ref[...]
    k_ref[...] = jnp.dot(s, wk_ref[...], preferred_element_type=_f32) + bk_ref[...]
    v_ref[...] = jnp.dot(s, wv_ref[...], preferred_element_type=_f32) + bv_ref[...]


def _tca(S_t, Wk, bk, Wv, bv):
    row = lambda i: (i, 0)
    full = lambda i: (0, 0)
    rspec = pl.BlockSpec((RB, D), row)
    return pl.pallas_call(
        _tca_body,
        grid=(B // RB,),
        in_specs=[
            rspec,
            pl.BlockSpec((D, D), full), pl.BlockSpec((1, D), full),
            pl.BlockSpec((D, D), full), pl.BlockSpec((1, D), full),
        ],
        out_specs=[rspec, rspec],
        out_shape=[
            jax.ShapeDtypeStruct((B, D), _f32),
            jax.ShapeDtypeStruct((B, D), _f32),
        ],
    )(S_t, Wk, bk.reshape(1, D), Wv, bv.reshape(1, D))


def _tcc_body(s_ref, k_ref, sk_ref, sv_ref, w1a_ref, w1b_ref, b1_ref,
              w2_ref, b2_ref, d_ref):
    s = s_ref[...]
    diff = k_ref[...] - sk_ref[...]
    match = jnp.exp(-jnp.mean(diff * diff, axis=-1, keepdims=True))
    rv = sv_ref[...] * match
    h = jnp.dot(s, w1a_ref[...], preferred_element_type=_f32)
    h = h + jnp.dot(rv, w1b_ref[...], preferred_element_type=_f32)
    h = jax.nn.relu(h + b1_ref[...])
    d_ref[...] = jnp.dot(h, w2_ref[...], preferred_element_type=_f32) + b2_ref[...]


def _tcc(S_t, k, stored_k, stored_v, W1, b1, W2, b2):
    w1a, w1b = W1[:D], W1[D:]
    row = lambda i: (i, 0)
    full = lambda i: (0, 0)
    rspec = pl.BlockSpec((RB, D), row)
    return pl.pallas_call(
        _tcc_body,
        grid=(B // RB,),
        in_specs=[
            rspec, rspec, rspec, rspec,
            pl.BlockSpec((D, H), full), pl.BlockSpec((D, H), full),
            pl.BlockSpec((1, H), full),
            pl.BlockSpec((H, D), full), pl.BlockSpec((1, D), full),
        ],
        out_specs=rspec,
        out_shape=jax.ShapeDtypeStruct((B, D), _f32),
    )(S_t, k, stored_k, stored_v, w1a, w1b, b1.reshape(1, H),
      W2, b2.reshape(1, D))


RT = 5000


def _tcb_body(mk_ref, mv_ref, nk_ref, nv_ref):
    nk_ref[...] = mk_ref[...]
    nv_ref[...] = mv_ref[...]


def _tcb(mem_keys, mem_vals):
    row = lambda i: (i, 0)
    rspec = pl.BlockSpec((RT, D), row)
    return pl.pallas_call(
        _tcb_body,
        grid=(M // RT,),
        in_specs=[rspec, rspec],
        out_specs=[rspec, rspec],
        out_shape=[
            jax.ShapeDtypeStruct((M, D), _f32),
            jax.ShapeDtypeStruct((M, D), _f32),
        ],
    )(mem_keys, mem_vals)


@jax.jit
def kernel(S_t, mem_keys, mem_vals, Wk, bk, Wv, bv, W1, b1, W2, b2, idx):
    stored_k, stored_v, postbl = _sc1(mem_keys, mem_vals, idx)
    nk, nv = _tcb(mem_keys, mem_vals)
    k, v = _tca(S_t, Wk, bk, Wv, bv)
    delta = _tcc(S_t, k, stored_k, stored_v, W1, b1, W2, b2)
    nk_ref = jax.new_ref(nk)
    nv_ref = jax.new_ref(nv)
    idx2 = idx.reshape(B // SUB, SUB)
    _sc2(idx2, postbl, k, v, nk_ref, nv_ref)
    return delta, nk_ref[...], nv_ref[...]


# speedup vs baseline: 1.0418x; 1.0361x over previous
import jax
import jax.numpy as jnp
from jax import lax
from jax.experimental import pallas as pl
from jax.experimental.pallas import tpu as pltpu
from jax.experimental.pallas import tpu_sc as plsc

B = 16384
D = 128
H = 128
M = 100000

NC = 2
NS = 16
NW = NC * NS
CHUNK = B // NW
SUB = 128
NSUB = CHUNK // SUB
STRIP = 3200
MPAD = NW * STRIP
NVREG = B // 16
PBITS = 14

_f32 = jnp.float32
_i32 = jnp.int32


def _sc_mesh():
    return plsc.VectorSubcoreMesh(core_axis_name="c", subcore_axis_name="s")


def _wid():
    return lax.axis_index("s") * NC + lax.axis_index("c")


def _sc1_body(memk_hbm, memv_hbm, idx_hbm,
              stored_k_hbm, stored_v_hbm, postbl_hbm,
              idxall_v, kb0, vb0, kb1, vb1, strip_v,
              semg, semw):
    wid = _wid()
    pltpu.sync_copy(idx_hbm, idxall_v)
    kb = (kb0, kb1)
    vb = (vb0, vb1)

    def fire_gather(p):
        base = (wid * NSUB + p) * SUB
        il = idxall_v.at[pl.ds(base, SUB)]
        return (pltpu.async_copy(memk_hbm.at[il], kb[p % 2], semg),
                pltpu.async_copy(memv_hbm.at[il], vb[p % 2], semg))

    def fire_wb(p):
        base = (wid * NSUB + p) * SUB
        return (pltpu.async_copy(kb[p % 2], stored_k_hbm.at[pl.ds(base, SUB)], semw),
                pltpu.async_copy(vb[p % 2], stored_v_hbm.at[pl.ds(base, SUB)], semw))

    lanes = lax.broadcasted_iota(_i32, (16,), 0)
    nxt = jnp.minimum(lanes + 1, 15)
    strip_lo = wid * STRIP

    UNROLL = 4

    def win_it(tb, _):
        for u in range(UNROLL):
            t = tb * UNROLL + u
            iv = idxall_v[pl.ds(t * 16, 16)]
            key = (iv << PBITS) | (t * 16 + lanes)
            key = lax.sort(key)
            nkey = jnp.take_along_axis(key, nxt, axis=0)
            slot = key >> PBITS
            keep = (lanes == 15) | (slot != (nkey >> PBITS))
            loc = slot - strip_lo
            inr = (loc >= 0) & (loc < STRIP)
            safe = jnp.where(inr, loc, 0)
            plsc.store_scatter(strip_v, [safe], key & (2**PBITS - 1),
                               mask=keep & inr)
        return _

    WSLICE = NVREG // UNROLL // NSUB
    g = fire_gather(0)
    wbs = []
    for p in range(NSUB):
        lax.fori_loop(p * WSLICE, (p + 1) * WSLICE, win_it, 0)
        g[0].wait()
        g[1].wait()
        if p + 1 < NSUB:
            if p >= 1:
                wbs[p - 1][0].wait()
                wbs[p - 1][1].wait()
            g = fire_gather(p + 1)
        wbs.append(fire_wb(p))

    pltpu.sync_copy(strip_v, postbl_hbm.at[pl.ds(strip_lo, STRIP)])
    wbs[-2][0].wait()
    wbs[-2][1].wait()
    wbs[-1][0].wait()
    wbs[-1][1].wait()


def _sc1(mem_keys, mem_vals, idx):
    kern = pl.kernel(
        _sc1_body,
        out_type=(
            jax.ShapeDtypeStruct((B, D), _f32),
            jax.ShapeDtypeStruct((B, D), _f32),
            jax.ShapeDtypeStruct((MPAD,), _i32),
        ),
        mesh=_sc_mesh(),
        compiler_params=pltpu.CompilerParams(needs_layout_passes=False),
        cost_estimate=pl.CostEstimate(
            flops=0, transcendentals=0, bytes_accessed=40_000_000),
        scratch_types=(
            pltpu.VMEM((B,), _i32),
            pltpu.VMEM((SUB, D), _f32),
            pltpu.VMEM((SUB, D), _f32),
            pltpu.VMEM((SUB, D), _f32),
            pltpu.VMEM((SUB, D), _f32),
            pltpu.VMEM((STRIP,), _i32),
            pltpu.SemaphoreType.DMA,
            pltpu.SemaphoreType.DMA,
        ),
    )
    return kern(mem_keys, mem_vals, idx)


def _sc2_body(idx2_hbm, postbl_hbm, k_hbm, v_hbm, nk_hbm, nv_hbm,
              idxc_v, wq_v, kb0, vb0, kb1, vb1, semw, semg, sems):
    wid = _wid()
    kb = (kb0, kb1)
    vb = (vb0, vb1)

    pltpu.sync_copy(idx2_hbm.at[pl.ds(wid * NSUB, NSUB)], idxc_v)
    wq = [pltpu.async_copy(postbl_hbm.at[idxc_v.at[p]], wq_v.at[p], semw)
          for p in range(NSUB)]
    for c in wq:
        c.wait()

    def fire_gather(p):
        return (pltpu.async_copy(k_hbm.at[wq_v.at[p]], kb[p % 2], semg),
                pltpu.async_copy(v_hbm.at[wq_v.at[p]], vb[p % 2], semg))

    def fire_scatter(p):
        return (pltpu.async_copy(kb[p % 2], nk_hbm.at[idxc_v.at[p]], sems),
                pltpu.async_copy(vb[p % 2], nv_hbm.at[idxc_v.at[p]], sems))

    g = fire_gather(0)
    scs = []
    for p in range(NSUB):
        g[0].wait()
        g[1].wait()
        if p + 1 < NSUB:
            if p >= 1:
                scs[p - 1][0].wait()
                scs[p - 1][1].wait()
            g = fire_gather(p + 1)
        scs.append(fire_scatter(p))
    scs[-2][0].wait()
    scs[-2][1].wait()
    scs[-1][0].wait()
    scs[-1][1].wait()


def _sc2(idx2, postbl, k, v, nk_ref, nv_ref):
    kern = pl.kernel(
        _sc2_body,
        out_type=(),
        mesh=_sc_mesh(),
        compiler_params=pltpu.CompilerParams(needs_layout_passes=False),
        cost_estimate=pl.CostEstimate(
            flops=0, transcendentals=0, bytes_accessed=34_000_000),
        scratch_types=(
            pltpu.VMEM((NSUB, SUB), _i32),
            pltpu.VMEM((NSUB, SUB), _i32),
            pltpu.VMEM((SUB, D), _f32),
            pltpu.VMEM((SUB, D), _f32),
            pltpu.VMEM((SUB, D), _f32),
            pltpu.VMEM((SUB, D), _f32),
            pltpu.SemaphoreType.DMA,
            pltpu.SemaphoreType.DMA,
            pltpu.SemaphoreType.DMA,
        ),
    )
    kern(idx2, postbl, k, v, nk_ref, nv_ref)


RB = 1024


def _tcac_body(s_ref, sk_ref, sv_ref, wk_ref, bk_ref, wv_ref, bv_ref,
               w1a_ref, w1b_ref, b1_ref, w2_ref, b2_ref,
               k_ref, v_ref, d_ref):
    s = s_ref[...]
    kk = jnp.dot(s, wk_ref[...], preferred_element_type=_f32) + bk_ref[...]
    vv = jnp.dot(s, wv_ref[...], preferred_element_type=_f32) + bv_ref[...]
    k_ref[...] = kk
    v_ref[...] = vv
    diff = kk - sk_ref[...]
    match = jnp.exp(-jnp.mean(diff * diff, axis=-1, keepdims=True))
    rv = sv_ref[...] * match
    h = jnp.dot(s, w1a_ref[...], preferred_element_type=_f32)
    h = h + jnp.dot(rv, w1b_ref[...], preferred_element_type=_f32)
    h = jax.nn.relu(h + b1_ref[...])
    d_ref[...] = jnp.dot(h, w2_ref[...], preferred_element_type=_f32) + b2_ref[...]


def _tcac(S_t, stored_k, stored_v, Wk, bk, Wv, bv, W1, b1, W2, b2):
    w1a, w1b = W1[:D], W1[D:]
    row = lambda i: (i, 0)
    full = lambda i: (0, 0)
    rspec = pl.BlockSpec((RB, D), row)
    return pl.pallas_call(
        _tcac_body,
        grid=(B // RB,),
        in_specs=[
            rspec, rspec, rspec,
            pl.BlockSpec((D, D), full), pl.BlockSpec((1, D), full),
            pl.BlockSpec((D, D), full), pl.BlockSpec((1, D), full),
            pl.BlockSpec((D, H), full), pl.BlockSpec((D, H), full),
            pl.BlockSpec((1, H), full),
            pl.BlockSpec((H, D), full), pl.BlockSpec((1, D), full),
        ],
        out_specs=[rspec, rspec, rspec],
        out_shape=[
            jax.ShapeDtypeStruct((B, D), _f32),
            jax.ShapeDtypeStruct((B, D), _f32),
            jax.ShapeDtypeStruct((B, D), _f32),
        ],
    )(S_t, stored_k, stored_v, Wk, bk.reshape(1, D), Wv, bv.reshape(1, D),
      w1a, w1b, b1.reshape(1, H), W2, b2.reshape(1, D))


RT = 5000


def _tcb_body(mk_ref, mv_ref, nk_ref, nv_ref):
    nk_ref[...] = mk_ref[...]
    nv_ref[...] = mv_ref[...]


def _tcb(mem_keys, mem_vals):
    row = lambda i: (i, 0)
    rspec = pl.BlockSpec((RT, D), row)
    return pl.pallas_call(
        _tcb_body,
        grid=(M // RT,),
        in_specs=[rspec, rspec],
        out_specs=[rspec, rspec],
        out_shape=[
            jax.ShapeDtypeStruct((M, D), _f32),
            jax.ShapeDtypeStruct((M, D), _f32),
        ],
    )(mem_keys, mem_vals)


@jax.jit
def kernel(S_t, mem_keys, mem_vals, Wk, bk, Wv, bv, W1, b1, W2, b2, idx):
    stored_k, stored_v, postbl = _sc1(mem_keys, mem_vals, idx)
    nk, nv = _tcb(mem_keys, mem_vals)
    k, v, delta = _tcac(S_t, stored_k, stored_v, Wk, bk, Wv, bv, W1, b1, W2, b2)
    nk_ref = jax.new_ref(nk)
    nv_ref = jax.new_ref(nv)
    idx2 = idx.reshape(B // SUB, SUB)
    _sc2(idx2, postbl, k, v, nk_ref, nv_ref)
    return delta, nk_ref[...], nv_ref[...]


# speedup vs baseline: 1.0444x; 1.0025x over previous
import jax
import jax.numpy as jnp
from jax import lax
from jax.experimental import pallas as pl
from jax.experimental.pallas import tpu as pltpu
from jax.experimental.pallas import tpu_sc as plsc

B = 16384
D = 128
H = 128
M = 100000

NC = 2
NS = 16
NW = NC * NS
CHUNK = B // NW
SUB = 128
NSUB = CHUNK // SUB
STRIP = 3200
MPAD = NW * STRIP
NVREG = B // 16
PBITS = 14

_f32 = jnp.float32
_i32 = jnp.int32


def _sc_mesh():
    return plsc.VectorSubcoreMesh(core_axis_name="c", subcore_axis_name="s")


def _wid():
    return lax.axis_index("s") * NC + lax.axis_index("c")


def _sc1_body(memk_hbm, memv_hbm, idx_hbm,
              stored_k_hbm, stored_v_hbm, postbl_hbm,
              idxall_v, kb0, vb0, kb1, vb1, strip_v,
              semg, semw):
    wid = _wid()
    pltpu.sync_copy(idx_hbm, idxall_v)
    kb = (kb0, kb1)
    vb = (vb0, vb1)

    def fire_gather(p):
        base = (wid * NSUB + p) * SUB
        il = idxall_v.at[pl.ds(base, SUB)]
        return (pltpu.async_copy(memk_hbm.at[il], kb[p % 2], semg),
                pltpu.async_copy(memv_hbm.at[il], vb[p % 2], semg))

    def fire_wb(p):
        base = (wid * NSUB + p) * SUB
        return (pltpu.async_copy(kb[p % 2], stored_k_hbm.at[pl.ds(base, SUB)], semw),
                pltpu.async_copy(vb[p % 2], stored_v_hbm.at[pl.ds(base, SUB)], semw))

    lanes = lax.broadcasted_iota(_i32, (16,), 0)
    nxt = jnp.minimum(lanes + 1, 15)
    strip_lo = wid * STRIP

    UNROLL = 4

    def win_it(tb, _):
        for u in range(UNROLL):
            t = tb * UNROLL + u
            iv = idxall_v[pl.ds(t * 16, 16)]
            key = (iv << PBITS) | (t * 16 + lanes)
            key = lax.sort(key)
            nkey = jnp.take_along_axis(key, nxt, axis=0)
            slot = key >> PBITS
            keep = (lanes == 15) | (slot != (nkey >> PBITS))
            loc = slot - strip_lo
            inr = (loc >= 0) & (loc < STRIP)
            safe = jnp.where(inr, loc, 0)
            plsc.store_scatter(strip_v, [safe], key & (2**PBITS - 1),
                               mask=keep & inr)
        return _

    WSLICE = NVREG // UNROLL // NSUB
    g = fire_gather(0)
    wbs = []
    for p in range(NSUB):
        lax.fori_loop(p * WSLICE, (p + 1) * WSLICE, win_it, 0)
        g[0].wait()
        g[1].wait()
        if p + 1 < NSUB:
            if p >= 1:
                wbs[p - 1][0].wait()
                wbs[p - 1][1].wait()
            g = fire_gather(p + 1)
        wbs.append(fire_wb(p))

    pltpu.sync_copy(strip_v, postbl_hbm.at[pl.ds(strip_lo, STRIP)])
    wbs[-2][0].wait()
    wbs[-2][1].wait()
    wbs[-1][0].wait()
    wbs[-1][1].wait()


def _sc1(mem_keys, mem_vals, idx):
    kern = pl.kernel(
        _sc1_body,
        out_type=(
            jax.ShapeDtypeStruct((B, D), _f32),
            jax.ShapeDtypeStruct((B, D), _f32),
            jax.ShapeDtypeStruct((MPAD,), _i32),
        ),
        mesh=_sc_mesh(),
        compiler_params=pltpu.CompilerParams(needs_layout_passes=False),
        cost_estimate=pl.CostEstimate(
            flops=0, transcendentals=0, bytes_accessed=40_000_000),
        scratch_types=(
            pltpu.VMEM((B,), _i32),
            pltpu.VMEM((SUB, D), _f32),
            pltpu.VMEM((SUB, D), _f32),
            pltpu.VMEM((SUB, D), _f32),
            pltpu.VMEM((SUB, D), _f32),
            pltpu.VMEM((STRIP,), _i32),
            pltpu.SemaphoreType.DMA,
            pltpu.SemaphoreType.DMA,
        ),
    )
    return kern(mem_keys, mem_vals, idx)


def _sc2_body(idx2_hbm, postbl_hbm, k_hbm, v_hbm, nk_hbm, nv_hbm,
              idxc_v, wq_v, kb0, vb0, kb1, vb1, semw, semg, sems):
    wid = _wid()
    kb = (kb0, kb1)
    vb = (vb0, vb1)

    pltpu.sync_copy(idx2_hbm.at[pl.ds(wid * NSUB, NSUB)], idxc_v)
    wq = [pltpu.async_copy(postbl_hbm.at[idxc_v.at[p]], wq_v.at[p], semw)
          for p in range(NSUB)]
    for c in wq:
        c.wait()

    def fire_gather(p):
        return (pltpu.async_copy(k_hbm.at[wq_v.at[p]], kb[p % 2], semg),
                pltpu.async_copy(v_hbm.at[wq_v.at[p]], vb[p % 2], semg))

    def fire_scatter(p):
        return (pltpu.async_copy(kb[p % 2], nk_hbm.at[idxc_v.at[p]], sems),
                pltpu.async_copy(vb[p % 2], nv_hbm.at[idxc_v.at[p]], sems))

    g = fire_gather(0)
    scs = []
    for p in range(NSUB):
        g[0].wait()
        g[1].wait()
        if p + 1 < NSUB:
            if p >= 1:
                scs[p - 1][0].wait()
                scs[p - 1][1].wait()
            g = fire_gather(p + 1)
        scs.append(fire_scatter(p))
    scs[-2][0].wait()
    scs[-2][1].wait()
    scs[-1][0].wait()
    scs[-1][1].wait()


def _sc2(idx2, postbl, k, v, nk_ref, nv_ref):
    kern = pl.kernel(
        _sc2_body,
        out_type=(),
        mesh=_sc_mesh(),
        compiler_params=pltpu.CompilerParams(needs_layout_passes=False),
        cost_estimate=pl.CostEstimate(
            flops=0, transcendentals=0, bytes_accessed=34_000_000),
        scratch_types=(
            pltpu.VMEM((NSUB, SUB), _i32),
            pltpu.VMEM((NSUB, SUB), _i32),
            pltpu.VMEM((SUB, D), _f32),
            pltpu.VMEM((SUB, D), _f32),
            pltpu.VMEM((SUB, D), _f32),
            pltpu.VMEM((SUB, D), _f32),
            pltpu.SemaphoreType.DMA,
            pltpu.SemaphoreType.DMA,
            pltpu.SemaphoreType.DMA,
        ),
    )
    kern(idx2, postbl, k, v, nk_ref, nv_ref)


RB = 1024


def _tcac_body(s_ref, sk_ref, sv_ref, wk_ref, bk_ref, wv_ref, bv_ref,
               w1a_ref, w1b_ref, b1_ref, w2_ref, b2_ref,
               k_ref, v_ref, d_ref):
    s = s_ref[...]
    kk = jnp.dot(s, wk_ref[...], preferred_element_type=_f32) + bk_ref[...]
    vv = jnp.dot(s, wv_ref[...], preferred_element_type=_f32) + bv_ref[...]
    k_ref[...] = kk
    v_ref[...] = vv
    diff = kk - sk_ref[...]
    match = jnp.exp(-jnp.mean(diff * diff, axis=-1, keepdims=True))
    rv = sv_ref[...] * match
    h = jnp.dot(s, w1a_ref[...], preferred_element_type=_f32)
    h = h + jnp.dot(rv, w1b_ref[...], preferred_element_type=_f32)
    h = jax.nn.relu(h + b1_ref[...])
    d_ref[...] = jnp.dot(h, w2_ref[...], preferred_element_type=_f32) + b2_ref[...]


def _tcac(S_t, stored_k, stored_v, Wk, bk, Wv, bv, W1, b1, W2, b2):
    w1a, w1b = W1[:D], W1[D:]
    row = lambda i: (i, 0)
    full = lambda i: (0, 0)
    rspec = pl.BlockSpec((RB, D), row)
    return pl.pallas_call(
        _tcac_body,
        grid=(B // RB,),
        in_specs=[
            rspec, rspec, rspec,
            pl.BlockSpec((D, D), full), pl.BlockSpec((1, D), full),
            pl.BlockSpec((D, D), full), pl.BlockSpec((1, D), full),
            pl.BlockSpec((D, H), full), pl.BlockSpec((D, H), full),
            pl.BlockSpec((1, H), full),
            pl.BlockSpec((H, D), full), pl.BlockSpec((1, D), full),
        ],
        out_specs=[rspec, rspec, rspec],
        out_shape=[
            jax.ShapeDtypeStruct((B, D), _f32),
            jax.ShapeDtypeStruct((B, D), _f32),
            jax.ShapeDtypeStruct((B, D), _f32),
        ],
    )(S_t, stored_k, stored_v, Wk, bk.reshape(1, D), Wv, bv.reshape(1, D),
      w1a, w1b, b1.reshape(1, H), W2, b2.reshape(1, D))


RT = 10000


def _tcb_body(mk_ref, mv_ref, nk_ref, nv_ref):
    nk_ref[...] = mk_ref[...]
    nv_ref[...] = mv_ref[...]


def _tcb(mem_keys, mem_vals):
    row = lambda i: (i, 0)
    rspec = pl.BlockSpec((RT, D), row)
    return pl.pallas_call(
        _tcb_body,
        grid=(M // RT,),
        in_specs=[rspec, rspec],
        out_specs=[rspec, rspec],
        out_shape=[
            jax.ShapeDtypeStruct((M, D), _f32),
            jax.ShapeDtypeStruct((M, D), _f32),
        ],
    )(mem_keys, mem_vals)


@jax.jit
def kernel(S_t, mem_keys, mem_vals, Wk, bk, Wv, bv, W1, b1, W2, b2, idx):
    stored_k, stored_v, postbl = _sc1(mem_keys, mem_vals, idx)
    nk, nv = _tcb(mem_keys, mem_vals)
    k, v, delta = _tcac(S_t, stored_k, stored_v, Wk, bk, Wv, bv, W1, b1, W2, b2)
    nk_ref = jax.new_ref(nk)
    nv_ref = jax.new_ref(nv)
    idx2 = idx.reshape(B // SUB, SUB)
    _sc2(idx2, postbl, k, v, nk_ref, nv_ref)
    return delta, nk_ref[...], nv_ref[...]


# speedup vs baseline: 1.0903x; 1.0439x over previous
import jax
import jax.numpy as jnp
from jax import lax
from jax.experimental import pallas as pl
from jax.experimental.pallas import tpu as pltpu
from jax.experimental.pallas import tpu_sc as plsc

B = 16384
D = 128
H = 128
M = 100000

NC = 2
NS = 16
NW = NC * NS
CHUNK = B // NW
SUB = 128
NSUB = CHUNK // SUB
STRIP = 3200
MPAD = NW * STRIP
NVREG = B // 16
PBITS = 14

_f32 = jnp.float32
_i32 = jnp.int32


def _sc_mesh():
    return plsc.VectorSubcoreMesh(core_axis_name="c", subcore_axis_name="s")


def _wid():
    return lax.axis_index("s") * NC + lax.axis_index("c")


def _sc1_body(memk_hbm, memv_hbm, idx_hbm,
              stored_k_hbm, stored_v_hbm, postbl_hbm,
              idxall_v, kb0, vb0, kb1, vb1, strip_v,
              semg, semw):
    wid = _wid()
    pltpu.sync_copy(idx_hbm, idxall_v)
    kb = (kb0, kb1)
    vb = (vb0, vb1)

    def fire_gather(p):
        base = (wid * NSUB + p) * SUB
        il = idxall_v.at[pl.ds(base, SUB)]
        return (pltpu.async_copy(memk_hbm.at[il], kb[p % 2], semg),
                pltpu.async_copy(memv_hbm.at[il], vb[p % 2], semg))

    def fire_wb(p):
        base = (wid * NSUB + p) * SUB
        return (pltpu.async_copy(kb[p % 2], stored_k_hbm.at[pl.ds(base, SUB)], semw),
                pltpu.async_copy(vb[p % 2], stored_v_hbm.at[pl.ds(base, SUB)], semw))

    lanes = lax.broadcasted_iota(_i32, (16,), 0)
    nxt = jnp.minimum(lanes + 1, 15)
    strip_lo = wid * STRIP

    UNROLL = 4

    def win_it(tb, _):
        for u in range(UNROLL):
            t = tb * UNROLL + u
            iv = idxall_v[pl.ds(t * 16, 16)]
            key = (iv << PBITS) | (t * 16 + lanes)
            key = lax.sort(key)
            nkey = jnp.take_along_axis(key, nxt, axis=0)
            slot = key >> PBITS
            keep = (lanes == 15) | (slot != (nkey >> PBITS))
            loc = slot - strip_lo
            inr = (loc >= 0) & (loc < STRIP)
            safe = jnp.where(inr, loc, 0)
            plsc.store_scatter(strip_v, [safe], key & (2**PBITS - 1),
                               mask=keep & inr)
        return _

    WSLICE = NVREG // UNROLL // NSUB
    g = fire_gather(0)
    wbs = []
    for p in range(NSUB):
        lax.fori_loop(p * WSLICE, (p + 1) * WSLICE, win_it, 0)
        g[0].wait()
        g[1].wait()
        if p + 1 < NSUB:
            if p >= 1:
                wbs[p - 1][0].wait()
                wbs[p - 1][1].wait()
            g = fire_gather(p + 1)
        wbs.append(fire_wb(p))

    pltpu.sync_copy(strip_v, postbl_hbm.at[pl.ds(strip_lo, STRIP)])
    wbs[-2][0].wait()
    wbs[-2][1].wait()
    wbs[-1][0].wait()
    wbs[-1][1].wait()


def _sc1(mem_keys, mem_vals, idx):
    kern = pl.kernel(
        _sc1_body,
        out_type=(
            jax.ShapeDtypeStruct((B, D), _f32),
            jax.ShapeDtypeStruct((B, D), _f32),
            jax.ShapeDtypeStruct((MPAD,), _i32),
        ),
        mesh=_sc_mesh(),
        compiler_params=pltpu.CompilerParams(needs_layout_passes=False),
        cost_estimate=pl.CostEstimate(
            flops=0, transcendentals=0, bytes_accessed=40_000_000),
        scratch_types=(
            pltpu.VMEM((B,), _i32),
            pltpu.VMEM((SUB, D), _f32),
            pltpu.VMEM((SUB, D), _f32),
            pltpu.VMEM((SUB, D), _f32),
            pltpu.VMEM((SUB, D), _f32),
            pltpu.VMEM((STRIP,), _i32),
            pltpu.SemaphoreType.DMA,
            pltpu.SemaphoreType.DMA,
        ),
    )
    return kern(mem_keys, mem_vals, idx)


def _sc2_body(idx2_hbm, postbl_hbm, k_hbm, v_hbm, nk_hbm, nv_hbm,
              idxc_v, wq_v, kb0, vb0, kb1, vb1, semw, semg, sems):
    wid = _wid()
    kb = (kb0, kb1)
    vb = (vb0, vb1)

    pltpu.sync_copy(idx2_hbm.at[pl.ds(wid * NSUB, NSUB)], idxc_v)
    wq = [pltpu.async_copy(postbl_hbm.at[idxc_v.at[p]], wq_v.at[p], semw)
          for p in range(NSUB)]
    for c in wq:
        c.wait()

    def fire_gather(p):
        return (pltpu.async_copy(k_hbm.at[wq_v.at[p]], kb[p % 2], semg),
                pltpu.async_copy(v_hbm.at[wq_v.at[p]], vb[p % 2], semg))

    def fire_scatter(p):
        return (pltpu.async_copy(kb[p % 2], nk_hbm.at[idxc_v.at[p]], sems),
                pltpu.async_copy(vb[p % 2], nv_hbm.at[idxc_v.at[p]], sems))

    g = fire_gather(0)
    scs = []
    for p in range(NSUB):
        g[0].wait()
        g[1].wait()
        if p + 1 < NSUB:
            if p >= 1:
                scs[p - 1][0].wait()
                scs[p - 1][1].wait()
            g = fire_gather(p + 1)
        scs.append(fire_scatter(p))
    scs[-2][0].wait()
    scs[-2][1].wait()
    scs[-1][0].wait()
    scs[-1][1].wait()


def _sc2(idx2, postbl, k, v, nk_ref, nv_ref):
    kern = pl.kernel(
        _sc2_body,
        out_type=(),
        mesh=_sc_mesh(),
        compiler_params=pltpu.CompilerParams(needs_layout_passes=False),
        cost_estimate=pl.CostEstimate(
            flops=0, transcendentals=0, bytes_accessed=34_000_000),
        scratch_types=(
            pltpu.VMEM((NSUB, SUB), _i32),
            pltpu.VMEM((NSUB, SUB), _i32),
            pltpu.VMEM((SUB, D), _f32),
            pltpu.VMEM((SUB, D), _f32),
            pltpu.VMEM((SUB, D), _f32),
            pltpu.VMEM((SUB, D), _f32),
            pltpu.SemaphoreType.DMA,
            pltpu.SemaphoreType.DMA,
            pltpu.SemaphoreType.DMA,
        ),
    )
    kern(idx2, postbl, k, v, nk_ref, nv_ref)


RB = 2048


def _tcac_body(s_ref, sk_ref, sv_ref, wk_ref, bk_ref, wv_ref, bv_ref,
               w1a_ref, w1b_ref, b1_ref, w2_ref, b2_ref,
               k_ref, v_ref, d_ref):
    s = s_ref[...]
    kk = jnp.dot(s, wk_ref[...], preferred_element_type=_f32) + bk_ref[...]
    vv = jnp.dot(s, wv_ref[...], preferred_element_type=_f32) + bv_ref[...]
    k_ref[...] = kk
    v_ref[...] = vv
    diff = kk - sk_ref[...]
    match = jnp.exp(-jnp.mean(diff * diff, axis=-1, keepdims=True))
    rv = sv_ref[...] * match
    h = jnp.dot(s, w1a_ref[...], preferred_element_type=_f32)
    h = h + jnp.dot(rv, w1b_ref[...], preferred_element_type=_f32)
    h = jax.nn.relu(h + b1_ref[...])
    d_ref[...] = jnp.dot(h, w2_ref[...], preferred_element_type=_f32) + b2_ref[...]


def _tcac(S_t, stored_k, stored_v, Wk, bk, Wv, bv, W1, b1, W2, b2):
    w1a, w1b = W1[:D], W1[D:]
    row = lambda i: (i, 0)
    full = lambda i: (0, 0)
    rspec = pl.BlockSpec((RB, D), row)
    return pl.pallas_call(
        _tcac_body,
        grid=(B // RB,),
        in_specs=[
            rspec, rspec, rspec,
            pl.BlockSpec((D, D), full), pl.BlockSpec((1, D), full),
            pl.BlockSpec((D, D), full), pl.BlockSpec((1, D), full),
            pl.BlockSpec((D, H), full), pl.BlockSpec((D, H), full),
            pl.BlockSpec((1, H), full),
            pl.BlockSpec((H, D), full), pl.BlockSpec((1, D), full),
        ],
        out_specs=[rspec, rspec, rspec],
        out_shape=[
            jax.ShapeDtypeStruct((B, D), _f32),
            jax.ShapeDtypeStruct((B, D), _f32),
            jax.ShapeDtypeStruct((B, D), _f32),
        ],
    )(S_t, stored_k, stored_v, Wk, bk.reshape(1, D), Wv, bv.reshape(1, D),
      w1a, w1b, b1.reshape(1, H), W2, b2.reshape(1, D))


RT = 10000


def _tcb_body(mk_ref, mv_ref, nk_ref, nv_ref):
    nk_ref[...] = mk_ref[...]
    nv_ref[...] = mv_ref[...]


def _tcb(mem_keys, mem_vals):
    row = lambda i: (i, 0)
    rspec = pl.BlockSpec((RT, D), row)
    return pl.pallas_call(
        _tcb_body,
        grid=(M // RT,),
        in_specs=[rspec, rspec],
        out_specs=[rspec, rspec],
        out_shape=[
            jax.ShapeDtypeStruct((M, D), _f32),
            jax.ShapeDtypeStruct((M, D), _f32),
        ],
    )(mem_keys, mem_vals)


@jax.jit
def kernel(S_t, mem_keys, mem_vals, Wk, bk, Wv, bv, W1, b1, W2, b2, idx):
    stored_k, stored_v, postbl = _sc1(mem_keys, mem_vals, idx)
    nk, nv = _tcb(mem_keys, mem_vals)
    k, v, delta = _tcac(S_t, stored_k, stored_v, Wk, bk, Wv, bv, W1, b1, W2, b2)
    nk_ref = jax.new_ref(nk)
    nv_ref = jax.new_ref(nv)
    idx2 = idx.reshape(B // SUB, SUB)
    _sc2(idx2, postbl, k, v, nk_ref, nv_ref)
    return delta, nk_ref[...], nv_ref[...]


# speedup vs baseline: 1.1030x; 1.0116x over previous
import jax
import jax.numpy as jnp
from jax import lax
from jax.experimental import pallas as pl
from jax.experimental.pallas import tpu as pltpu
from jax.experimental.pallas import tpu_sc as plsc

B = 16384
D = 128
H = 128
M = 100000

NC = 2
NS = 16
NW = NC * NS
CHUNK = B // NW
SUB = 128
NSUB = CHUNK // SUB
STRIP = 3200
MPAD = NW * STRIP
NVREG = B // 16
PBITS = 14

_f32 = jnp.float32
_i32 = jnp.int32


def _sc_mesh():
    return plsc.VectorSubcoreMesh(core_axis_name="c", subcore_axis_name="s")


def _wid():
    return lax.axis_index("s") * NC + lax.axis_index("c")


def _sc1_body(memk_hbm, memv_hbm, idx_hbm,
              stored_k_hbm, stored_v_hbm, postbl_hbm,
              idxall_v, kb0, vb0, kb1, vb1, strip_v,
              semg, semw):
    wid = _wid()
    pltpu.sync_copy(idx_hbm, idxall_v)
    kb = (kb0, kb1)
    vb = (vb0, vb1)

    def fire_gather(p):
        base = (wid * NSUB + p) * SUB
        il = idxall_v.at[pl.ds(base, SUB)]
        return (pltpu.async_copy(memk_hbm.at[il], kb[p % 2], semg),
                pltpu.async_copy(memv_hbm.at[il], vb[p % 2], semg))

    def fire_wb(p):
        base = (wid * NSUB + p) * SUB
        return (pltpu.async_copy(kb[p % 2], stored_k_hbm.at[pl.ds(base, SUB)], semw),
                pltpu.async_copy(vb[p % 2], stored_v_hbm.at[pl.ds(base, SUB)], semw))

    lanes = lax.broadcasted_iota(_i32, (16,), 0)
    nxt = jnp.minimum(lanes + 1, 15)
    strip_lo = wid * STRIP

    UNROLL = 4

    def win_it(tb, _):
        for u in range(UNROLL):
            t = tb * UNROLL + u
            iv = idxall_v[pl.ds(t * 16, 16)]
            key = (iv << PBITS) | (t * 16 + lanes)
            key = lax.sort(key)
            nkey = jnp.take_along_axis(key, nxt, axis=0)
            slot = key >> PBITS
            keep = (lanes == 15) | (slot != (nkey >> PBITS))
            loc = slot - strip_lo
            inr = (loc >= 0) & (loc < STRIP)
            safe = jnp.where(inr, loc, 0)
            plsc.store_scatter(strip_v, [safe], key & (2**PBITS - 1),
                               mask=keep & inr)
        return _

    WSLICE = NVREG // UNROLL // NSUB
    g = fire_gather(0)
    wbs = []
    for p in range(NSUB):
        lax.fori_loop(p * WSLICE, (p + 1) * WSLICE, win_it, 0)
        g[0].wait()
        g[1].wait()
        if p + 1 < NSUB:
            if p >= 1:
                wbs[p - 1][0].wait()
                wbs[p - 1][1].wait()
            g = fire_gather(p + 1)
        wbs.append(fire_wb(p))

    pltpu.sync_copy(strip_v, postbl_hbm.at[pl.ds(strip_lo, STRIP)])
    wbs[-2][0].wait()
    wbs[-2][1].wait()
    wbs[-1][0].wait()
    wbs[-1][1].wait()


def _sc1(mem_keys, mem_vals, idx):
    kern = pl.kernel(
        _sc1_body,
        out_type=(
            jax.ShapeDtypeStruct((B, D), _f32),
            jax.ShapeDtypeStruct((B, D), _f32),
            jax.ShapeDtypeStruct((MPAD,), _i32),
        ),
        mesh=_sc_mesh(),
        compiler_params=pltpu.CompilerParams(needs_layout_passes=False),
        cost_estimate=pl.CostEstimate(
            flops=0, transcendentals=0, bytes_accessed=40_000_000),
        scratch_types=(
            pltpu.VMEM((B,), _i32),
            pltpu.VMEM((SUB, D), _f32),
            pltpu.VMEM((SUB, D), _f32),
            pltpu.VMEM((SUB, D), _f32),
            pltpu.VMEM((SUB, D), _f32),
            pltpu.VMEM((STRIP,), _i32),
            pltpu.SemaphoreType.DMA,
            pltpu.SemaphoreType.DMA,
        ),
    )
    return kern(mem_keys, mem_vals, idx)


def _sc2_body(idx2_hbm, postbl_hbm, k_hbm, v_hbm, nk_hbm, nv_hbm,
              idxc_v, wq_v, kb0, vb0, kb1, vb1, semw, semg, sems):
    wid = _wid()
    kb = (kb0, kb1)
    vb = (vb0, vb1)

    pltpu.sync_copy(idx2_hbm.at[pl.ds(wid * NSUB, NSUB)], idxc_v)
    wq = [pltpu.async_copy(postbl_hbm.at[idxc_v.at[p]], wq_v.at[p], semw)
          for p in range(NSUB)]
    for c in wq:
        c.wait()

    def fire_gather(p):
        return (pltpu.async_copy(k_hbm.at[wq_v.at[p]], kb[p % 2], semg),
                pltpu.async_copy(v_hbm.at[wq_v.at[p]], vb[p % 2], semg))

    def fire_scatter(p):
        return (pltpu.async_copy(kb[p % 2], nk_hbm.at[idxc_v.at[p]], sems),
                pltpu.async_copy(vb[p % 2], nv_hbm.at[idxc_v.at[p]], sems))

    g = fire_gather(0)
    scs = []
    for p in range(NSUB):
        g[0].wait()
        g[1].wait()
        if p + 1 < NSUB:
            if p >= 1:
                scs[p - 1][0].wait()
                scs[p - 1][1].wait()
            g = fire_gather(p + 1)
        scs.append(fire_scatter(p))
    scs[-2][0].wait()
    scs[-2][1].wait()
    scs[-1][0].wait()
    scs[-1][1].wait()


def _sc2(idx2, postbl, k, v, nk_ref, nv_ref):
    kern = pl.kernel(
        _sc2_body,
        out_type=(),
        mesh=_sc_mesh(),
        compiler_params=pltpu.CompilerParams(needs_layout_passes=False),
        cost_estimate=pl.CostEstimate(
            flops=0, transcendentals=0, bytes_accessed=34_000_000),
        scratch_types=(
            pltpu.VMEM((NSUB, SUB), _i32),
            pltpu.VMEM((NSUB, SUB), _i32),
            pltpu.VMEM((SUB, D), _f32),
            pltpu.VMEM((SUB, D), _f32),
            pltpu.VMEM((SUB, D), _f32),
            pltpu.VMEM((SUB, D), _f32),
            pltpu.SemaphoreType.DMA,
            pltpu.SemaphoreType.DMA,
            pltpu.SemaphoreType.DMA,
        ),
    )
    kern(idx2, postbl, k, v, nk_ref, nv_ref)


RB = 4096


def _tcac_body(s_ref, sk_ref, sv_ref, wk_ref, bk_ref, wv_ref, bv_ref,
               w1a_ref, w1b_ref, b1_ref, w2_ref, b2_ref,
               k_ref, v_ref, d_ref):
    s = s_ref[...]
    kk = jnp.dot(s, wk_ref[...], preferred_element_type=_f32) + bk_ref[...]
    vv = jnp.dot(s, wv_ref[...], preferred_element_type=_f32) + bv_ref[...]
    k_ref[...] = kk
    v_ref[...] = vv
    diff = kk - sk_ref[...]
    match = jnp.exp(-jnp.mean(diff * diff, axis=-1, keepdims=True))
    rv = sv_ref[...] * match
    h = jnp.dot(s, w1a_ref[...], preferred_element_type=_f32)
    h = h + jnp.dot(rv, w1b_ref[...], preferred_element_type=_f32)
    h = jax.nn.relu(h + b1_ref[...])
    d_ref[...] = jnp.dot(h, w2_ref[...], preferred_element_type=_f32) + b2_ref[...]


def _tcac(S_t, stored_k, stored_v, Wk, bk, Wv, bv, W1, b1, W2, b2):
    w1a, w1b = W1[:D], W1[D:]
    row = lambda i: (i, 0)
    full = lambda i: (0, 0)
    rspec = pl.BlockSpec((RB, D), row)
    return pl.pallas_call(
        _tcac_body,
        grid=(B // RB,),
        in_specs=[
            rspec, rspec, rspec,
            pl.BlockSpec((D, D), full), pl.BlockSpec((1, D), full),
            pl.BlockSpec((D, D), full), pl.BlockSpec((1, D), full),
            pl.BlockSpec((D, H), full), pl.BlockSpec((D, H), full),
            pl.BlockSpec((1, H), full),
            pl.BlockSpec((H, D), full), pl.BlockSpec((1, D), full),
        ],
        out_specs=[rspec, rspec, rspec],
        out_shape=[
            jax.ShapeDtypeStruct((B, D), _f32),
            jax.ShapeDtypeStruct((B, D), _f32),
            jax.ShapeDtypeStruct((B, D), _f32),
        ],
    )(S_t, stored_k, stored_v, Wk, bk.reshape(1, D), Wv, bv.reshape(1, D),
      w1a, w1b, b1.reshape(1, H), W2, b2.reshape(1, D))


RT = 10000


def _tcb_body(mk_ref, mv_ref, nk_ref, nv_ref):
    nk_ref[...] = mk_ref[...]
    nv_ref[...] = mv_ref[...]


def _tcb(mem_keys, mem_vals):
    row = lambda i: (i, 0)
    rspec = pl.BlockSpec((RT, D), row)
    return pl.pallas_call(
        _tcb_body,
        grid=(M // RT,),
        in_specs=[rspec, rspec],
        out_specs=[rspec, rspec],
        out_shape=[
            jax.ShapeDtypeStruct((M, D), _f32),
            jax.ShapeDtypeStruct((M, D), _f32),
        ],
    )(mem_keys, mem_vals)


@jax.jit
def kernel(S_t, mem_keys, mem_vals, Wk, bk, Wv, bv, W1, b1, W2, b2, idx):
    stored_k, stored_v, postbl = _sc1(mem_keys, mem_vals, idx)
    nk, nv = _tcb(mem_keys, mem_vals)
    k, v, delta = _tcac(S_t, stored_k, stored_v, Wk, bk, Wv, bv, W1, b1, W2, b2)
    nk_ref = jax.new_ref(nk)
    nv_ref = jax.new_ref(nv)
    idx2 = idx.reshape(B // SUB, SUB)
    _sc2(idx2, postbl, k, v, nk_ref, nv_ref)
    return delta, nk_ref[...], nv_ref[...]


# speedup vs baseline: 1.1231x; 1.0182x over previous
import jax
import jax.numpy as jnp
from jax import lax
from jax.experimental import pallas as pl
from jax.experimental.pallas import tpu as pltpu
from jax.experimental.pallas import tpu_sc as plsc

B = 16384
D = 128
H = 128
M = 100000

NC = 2
NS = 16
NW = NC * NS
CHUNK = B // NW
SUB = 128
NSUB = CHUNK // SUB
STRIP = 3200
MPAD = NW * STRIP
NVREG = B // 16
PBITS = 14

_f32 = jnp.float32
_i32 = jnp.int32


def _sc_mesh():
    return plsc.VectorSubcoreMesh(core_axis_name="c", subcore_axis_name="s")


def _wid():
    return lax.axis_index("s") * NC + lax.axis_index("c")


def _sc1_body(memk_hbm, memv_hbm, idx_hbm,
              stored_k_hbm, stored_v_hbm, postbl_hbm,
              idxall_v, kb0, vb0, kb1, vb1, strip_v,
              semg, semw):
    wid = _wid()
    pltpu.sync_copy(idx_hbm, idxall_v)
    kb = (kb0, kb1)
    vb = (vb0, vb1)

    def fire_gather(p):
        base = (wid * NSUB + p) * SUB
        il = idxall_v.at[pl.ds(base, SUB)]
        return (pltpu.async_copy(memk_hbm.at[il], kb[p % 2], semg),
                pltpu.async_copy(memv_hbm.at[il], vb[p % 2], semg))

    def fire_wb(p):
        base = (wid * NSUB + p) * SUB
        return (pltpu.async_copy(kb[p % 2], stored_k_hbm.at[pl.ds(base, SUB)], semw),
                pltpu.async_copy(vb[p % 2], stored_v_hbm.at[pl.ds(base, SUB)], semw))

    lanes = lax.broadcasted_iota(_i32, (16,), 0)
    nxt = jnp.minimum(lanes + 1, 15)
    strip_lo = wid * STRIP

    UNROLL = 4

    def win_it(tb, _):
        for u in range(UNROLL):
            t = tb * UNROLL + u
            iv = idxall_v[pl.ds(t * 16, 16)]
            key = (iv << PBITS) | (t * 16 + lanes)
            key = lax.sort(key)
            nkey = jnp.take_along_axis(key, nxt, axis=0)
            slot = key >> PBITS
            keep = (lanes == 15) | (slot != (nkey >> PBITS))
            loc = slot - strip_lo
            inr = (loc >= 0) & (loc < STRIP)
            safe = jnp.where(inr, loc, 0)
            plsc.store_scatter(strip_v, [safe], key & (2**PBITS - 1),
                               mask=keep & inr)
        return _

    WSLICE = NVREG // UNROLL // NSUB
    g = fire_gather(0)
    wbs = []
    for p in range(NSUB):
        lax.fori_loop(p * WSLICE, (p + 1) * WSLICE, win_it, 0)
        g[0].wait()
        g[1].wait()
        if p + 1 < NSUB:
            if p >= 1:
                wbs[p - 1][0].wait()
                wbs[p - 1][1].wait()
            g = fire_gather(p + 1)
        wbs.append(fire_wb(p))

    pltpu.sync_copy(strip_v, postbl_hbm.at[pl.ds(strip_lo, STRIP)])
    wbs[-2][0].wait()
    wbs[-2][1].wait()
    wbs[-1][0].wait()
    wbs[-1][1].wait()


def _sc1(mem_keys, mem_vals, idx):
    kern = pl.kernel(
        _sc1_body,
        out_type=(
            jax.ShapeDtypeStruct((B, D), _f32),
            jax.ShapeDtypeStruct((B, D), _f32),
            jax.ShapeDtypeStruct((MPAD,), _i32),
        ),
        mesh=_sc_mesh(),
        compiler_params=pltpu.CompilerParams(needs_layout_passes=False),
        cost_estimate=pl.CostEstimate(
            flops=0, transcendentals=0, bytes_accessed=40_000_000),
        scratch_types=(
            pltpu.VMEM((B,), _i32),
            pltpu.VMEM((SUB, D), _f32),
            pltpu.VMEM((SUB, D), _f32),
            pltpu.VMEM((SUB, D), _f32),
            pltpu.VMEM((SUB, D), _f32),
            pltpu.VMEM((STRIP,), _i32),
            pltpu.SemaphoreType.DMA,
            pltpu.SemaphoreType.DMA,
        ),
    )
    return kern(mem_keys, mem_vals, idx)


def _sc2_body(idx2_hbm, postbl_hbm, k_hbm, v_hbm, nk_hbm, nv_hbm,
              idxc_v, wq_v, kb0, vb0, kb1, vb1, semw, semg, sems):
    wid = _wid()
    kb = (kb0, kb1)
    vb = (vb0, vb1)

    pltpu.sync_copy(idx2_hbm.at[pl.ds(wid * NSUB, NSUB)], idxc_v)
    wq = [pltpu.async_copy(postbl_hbm.at[idxc_v.at[p]], wq_v.at[p], semw)
          for p in range(NSUB)]
    for c in wq:
        c.wait()

    def fire_gather(p):
        return (pltpu.async_copy(k_hbm.at[wq_v.at[p]], kb[p % 2], semg),
                pltpu.async_copy(v_hbm.at[wq_v.at[p]], vb[p % 2], semg))

    def fire_scatter(p):
        return (pltpu.async_copy(kb[p % 2], nk_hbm.at[idxc_v.at[p]], sems),
                pltpu.async_copy(vb[p % 2], nv_hbm.at[idxc_v.at[p]], sems))

    g = fire_gather(0)
    scs = []
    for p in range(NSUB):
        g[0].wait()
        g[1].wait()
        if p + 1 < NSUB:
            if p >= 1:
                scs[p - 1][0].wait()
                scs[p - 1][1].wait()
            g = fire_gather(p + 1)
        scs.append(fire_scatter(p))
    scs[-2][0].wait()
    scs[-2][1].wait()
    scs[-1][0].wait()
    scs[-1][1].wait()


def _sc2(idx2, postbl, k, v, nk_ref, nv_ref):
    kern = pl.kernel(
        _sc2_body,
        out_type=(),
        mesh=_sc_mesh(),
        compiler_params=pltpu.CompilerParams(needs_layout_passes=False),
        cost_estimate=pl.CostEstimate(
            flops=0, transcendentals=0, bytes_accessed=34_000_000),
        scratch_types=(
            pltpu.VMEM((NSUB, SUB), _i32),
            pltpu.VMEM((NSUB, SUB), _i32),
            pltpu.VMEM((SUB, D), _f32),
            pltpu.VMEM((SUB, D), _f32),
            pltpu.VMEM((SUB, D), _f32),
            pltpu.VMEM((SUB, D), _f32),
            pltpu.SemaphoreType.DMA,
            pltpu.SemaphoreType.DMA,
            pltpu.SemaphoreType.DMA,
        ),
    )
    kern(idx2, postbl, k, v, nk_ref, nv_ref)


RB = 8192


def _tcac_body(s_ref, sk_ref, sv_ref, wk_ref, bk_ref, wv_ref, bv_ref,
               w1a_ref, w1b_ref, b1_ref, w2_ref, b2_ref,
               k_ref, v_ref, d_ref):
    s = s_ref[...]
    kk = jnp.dot(s, wk_ref[...], preferred_element_type=_f32) + bk_ref[...]
    vv = jnp.dot(s, wv_ref[...], preferred_element_type=_f32) + bv_ref[...]
    k_ref[...] = kk
    v_ref[...] = vv
    diff = kk - sk_ref[...]
    match = jnp.exp(-jnp.mean(diff * diff, axis=-1, keepdims=True))
    rv = sv_ref[...] * match
    h = jnp.dot(s, w1a_ref[...], preferred_element_type=_f32)
    h = h + jnp.dot(rv, w1b_ref[...], preferred_element_type=_f32)
    h = jax.nn.relu(h + b1_ref[...])
    d_ref[...] = jnp.dot(h, w2_ref[...], preferred_element_type=_f32) + b2_ref[...]


def _tcac(S_t, stored_k, stored_v, Wk, bk, Wv, bv, W1, b1, W2, b2):
    w1a, w1b = W1[:D], W1[D:]
    row = lambda i: (i, 0)
    full = lambda i: (0, 0)
    rspec = pl.BlockSpec((RB, D), row)
    return pl.pallas_call(
        _tcac_body,
        grid=(B // RB,),
        in_specs=[
            rspec, rspec, rspec,
            pl.BlockSpec((D, D), full), pl.BlockSpec((1, D), full),
            pl.BlockSpec((D, D), full), pl.BlockSpec((1, D), full),
            pl.BlockSpec((D, H), full), pl.BlockSpec((D, H), full),
            pl.BlockSpec((1, H), full),
            pl.BlockSpec((H, D), full), pl.BlockSpec((1, D), full),
        ],
        out_specs=[rspec, rspec, rspec],
        out_shape=[
            jax.ShapeDtypeStruct((B, D), _f32),
            jax.ShapeDtypeStruct((B, D), _f32),
            jax.ShapeDtypeStruct((B, D), _f32),
        ],
    )(S_t, stored_k, stored_v, Wk, bk.reshape(1, D), Wv, bv.reshape(1, D),
      w1a, w1b, b1.reshape(1, H), W2, b2.reshape(1, D))


RT = 10000


def _tcb_body(mk_ref, mv_ref, nk_ref, nv_ref):
    nk_ref[...] = mk_ref[...]
    nv_ref[...] = mv_ref[...]


def _tcb(mem_keys, mem_vals):
    row = lambda i: (i, 0)
    rspec = pl.BlockSpec((RT, D), row)
    return pl.pallas_call(
        _tcb_body,
        grid=(M // RT,),
        in_specs=[rspec, rspec],
        out_specs=[rspec, rspec],
        out_shape=[
            jax.ShapeDtypeStruct((M, D), _f32),
            jax.ShapeDtypeStruct((M, D), _f32),
        ],
    )(mem_keys, mem_vals)


@jax.jit
def kernel(S_t, mem_keys, mem_vals, Wk, bk, Wv, bv, W1, b1, W2, b2, idx):
    stored_k, stored_v, postbl = _sc1(mem_keys, mem_vals, idx)
    nk, nv = _tcb(mem_keys, mem_vals)
    k, v, delta = _tcac(S_t, stored_k, stored_v, Wk, bk, Wv, bv, W1, b1, W2, b2)
    nk_ref = jax.new_ref(nk)
    nv_ref = jax.new_ref(nv)
    idx2 = idx.reshape(B // SUB, SUB)
    _sc2(idx2, postbl, k, v, nk_ref, nv_ref)
    return delta, nk_ref[...], nv_ref[...]
